# baseline (speedup 1.0000x reference)
import functools

import jax
import jax.numpy as jnp
from jax import lax
from jax.experimental import pallas as pl
from jax.experimental.pallas import tpu as pltpu
from jax.experimental.pallas import tpu_sc as plsc

N_AGT = 4096
N_CTX = 8192
D = 128
K = 128
SLACK = 16
AB = 64
EPS = 1e-5
SENT = 1e9


def _gn_rows(x, g, b):
    m = jnp.mean(x, axis=-1, keepdims=True)
    ms = jnp.mean(x * x, axis=-1, keepdims=True)
    v = ms - m * m
    return (x - m) * lax.rsqrt(v + EPS) * g + b


def _qpart_body(agts_ref, qW_ref, qg_ref, qb_ref, W1q_ref, o_ref):
    x = agts_ref[...]
    q = lax.dot_general(x, qW_ref[...], (((1,), (1,)), ((), ())))
    q = jnp.maximum(_gn_rows(q, qg_ref[...], qb_ref[...]), 0.0)
    o_ref[...] = lax.dot_general(q, W1q_ref[...], (((1,), (1,)), ((), ())))


def _cpart_body(ctx_ref, W1c_ref, o_ref):
    o_ref[...] = lax.dot_general(ctx_ref[...], W1c_ref[...],
                                 (((1,), (1,)), ((), ())))


def _run_prework(agts, q_W, q_g, q_b, W1q, ctx, W1c):
    rb = min(1024, N_AGT, N_CTX)
    full = lambda i: (0, 0)
    qpart = pl.pallas_call(
        _qpart_body,
        grid=(N_AGT // rb,),
        in_specs=[
            pl.BlockSpec((rb, D), lambda i: (i, 0)),
            pl.BlockSpec((D, D), full),
            pl.BlockSpec((1, D), full),
            pl.BlockSpec((1, D), full),
            pl.BlockSpec((D, D), full),
        ],
        out_specs=pl.BlockSpec((rb, D), lambda i: (i, 0)),
        out_shape=jax.ShapeDtypeStruct((N_AGT, D), jnp.float32),
    )(agts, q_W, q_g.reshape(1, D), q_b.reshape(1, D), W1q)
    cpart = pl.pallas_call(
        _cpart_body,
        grid=(N_CTX // rb,),
        in_specs=[
            pl.BlockSpec((rb, D), lambda i: (i, 0)),
            pl.BlockSpec((D, D), full),
        ],
        out_specs=pl.BlockSpec((rb, D), lambda i: (i, 0)),
        out_shape=jax.ShapeDtypeStruct((N_CTX, D), jnp.float32),
    )(ctx, W1c)
    return qpart, cpart


def _sc_search_gather(ctx_x, ctx_y, agt_x, agt_y, th2v, cpart):
    info = plsc.get_sparse_core_info()
    NC, NS = info.num_cores, info.num_subcores
    NW = NC * NS
    A_PER = N_AGT // NW

    mesh = plsc.VectorSubcoreMesh(core_axis_name="c", subcore_axis_name="s")

    @functools.partial(
        pl.kernel,
        out_type=(
            jax.ShapeDtypeStruct((N_AGT, K), jnp.float32),
            jax.ShapeDtypeStruct((N_AGT, K), jnp.float32),
            jax.ShapeDtypeStruct((N_AGT, K, D), jnp.float32),
        ),
        mesh=mesh,
        compiler_params=pltpu.CompilerParams(needs_layout_passes=False),
        scratch_types=[
            pltpu.VMEM((N_CTX,), jnp.float32),
            pltpu.VMEM((N_CTX,), jnp.float32),
            pltpu.VMEM((A_PER,), jnp.float32),
            pltpu.VMEM((A_PER,), jnp.float32),
            pltpu.VMEM((16,), jnp.float32),
            pltpu.VMEM((K + SLACK,), jnp.int32),
            pltpu.VMEM((K,), jnp.int32),
            pltpu.VMEM((K + SLACK,), jnp.float32),
            pltpu.VMEM((K + SLACK,), jnp.float32),
            pltpu.VMEM((K, D), jnp.float32),
            pltpu.VMEM_SHARED((N_CTX, D), jnp.float32),
            pltpu.SemaphoreType.DMA,
        ],
    )
    def body(ctx_x_h, ctx_y_h, agt_x_h, agt_y_h, th2_h, cpart_h,
             dvx_h, dvy_h, ef_h,
             cx, cy, ax, ay, th2s, idxb, idx2, dvxb, dvyb, rows,
             shared, sem):
        sid = lax.axis_index("s")
        wid = sid * NC + lax.axis_index("c")
        base = wid * A_PER
        sl = N_CTX // NS
        pltpu.sync_copy(cpart_h.at[pl.ds(sid * sl, sl)],
                        shared.at[pl.ds(sid * sl, sl)])
        pltpu.sync_copy(ctx_x_h, cx)
        pltpu.sync_copy(ctx_y_h, cy)
        pltpu.sync_copy(agt_x_h.at[pl.ds(base, A_PER)], ax)
        pltpu.sync_copy(agt_y_h.at[pl.ds(base, A_PER)], ay)
        pltpu.sync_copy(th2_h, th2s)
        plsc.subcore_barrier()
        th2 = th2s[...]
        lanes = lax.iota(jnp.int32, 16)
        zf = jnp.zeros((16,), jnp.float32)
        zi = jnp.zeros((16,), jnp.int32)
        sentinel = jnp.full((16,), SENT, jnp.float32)

        def per_agent(a, carry):
            for t in range(K // 16):
                idxb[pl.ds(t * 16, 16)] = zi
                dvxb[pl.ds(t * 16, 16)] = sentinel
            a0 = (a // 16) * 16
            lane = a - a0
            axs = jnp.sum(jnp.where(lanes == lane, ax[pl.ds(a0, 16)], zf))
            ays = jnp.sum(jnp.where(lanes == lane, ay[pl.ds(a0, 16)], zf))
            axb = jnp.full((16,), axs)
            ayb = jnp.full((16,), ays)

            @plsc.parallel_loop(0, N_CTX // 16, unroll=4, carry=zi)
            def _chunks(c, o):
                dx = axb - cx[pl.ds(c * 16, 16)]
                dy = ayb - cy[pl.ds(c * 16, 16)]
                m = dx * dx + dy * dy <= th2
                cum = plsc.cumsum(m.astype(jnp.int32))
                pos = jnp.clip(o + cum - 1, 0, K + SLACK - 1)
                plsc.store_scatter(idxb, [pos], c * 16 + lanes, mask=m)
                plsc.store_scatter(dvxb, [pos], dx, mask=m)
                plsc.store_scatter(dvyb, [pos], dy, mask=m)
                return o + plsc.all_reduce_population_count(m)

            for t in range(K // 16):
                idx2[pl.ds(t * 16, 16)] = idxb[pl.ds(t * 16, 16)]
            g = base + a
            pltpu.async_copy(shared.at[idx2], rows, sem).wait()
            pltpu.sync_copy(rows, ef_h.at[g])
            pltpu.sync_copy(dvxb.at[pl.ds(0, K)], dvx_h.at[g])
            pltpu.sync_copy(dvyb.at[pl.ds(0, K)], dvy_h.at[g])
            return carry

        lax.fori_loop(0, A_PER, per_agent, 0)

    return body(ctx_x, ctx_y, agt_x, agt_y, th2v, cpart)


def _edge_body(ef_ref, dvx_ref, dvy_ref, qp_ref, agts_ref,
               w1x_ref, w1y_ref, b1d_ref, dW2_ref, dg2_ref, db2_ref,
               W1d_ref, cg1_ref, cb1_ref, cW2_ref,
               aW_ref, ng_ref, nb_ref, lW_ref, lg_ref, lb_ref, o_ref,
               S_ref):
    R = AB * K
    @pl.when(pl.program_id(0) == 0)
    def _():
        row_agent = lax.broadcasted_iota(jnp.int32, (R, AB), 0) // K
        col = lax.broadcasted_iota(jnp.int32, (R, AB), 1)
        S_ref[...] = (row_agent == col).astype(jnp.float32)
    S = S_ref[...]
    dvx = dvx_ref[...]
    dvy = dvy_ref[...]
    d1 = jnp.maximum(dvx * w1x_ref[...] + dvy * w1y_ref[...] + b1d_ref[...],
                     0.0)
    d2 = lax.dot_general(d1, dW2_ref[...], (((1,), (1,)), ((), ())))
    d2 = jnp.maximum(_gn_rows(d2, dg2_ref[...], db2_ref[...]), 0.0)
    z = lax.dot_general(d2, W1d_ref[...], (((1,), (1,)), ((), ())))
    z = z + ef_ref[...].reshape(R, D)
    z = z + lax.dot_general(S, qp_ref[...], (((1,), (0,)), ((), ())))
    h = jnp.maximum(_gn_rows(z, cg1_ref[...], cb1_ref[...]), 0.0)
    e = lax.dot_general(h, cW2_ref[...], (((1,), (1,)), ((), ())))
    e = jnp.where(dvx < SENT * 0.5, e, 0.0)
    msgs = lax.dot_general(S, e, (((0,), (0,)), ((), ())))
    res = agts_ref[...]
    a = lax.dot_general(res, aW_ref[...], (((1,), (1,)), ((), ()))) + msgs
    a = jnp.maximum(_gn_rows(a, ng_ref[...], nb_ref[...]), 0.0)
    a = lax.dot_general(a, lW_ref[...], (((1,), (1,)), ((), ())))
    a = _gn_rows(a, lg_ref[...], lb_ref[...])
    o_ref[...] = jnp.maximum(a + res, 0.0)


def _run_edge(ef, dvx, dvy, qpart, agts,
              w1x, w1y, b1d, dist_W2, dg2, db2,
              W1d, cg1, cb1, ctx_W2, agt_W, ng, nb, lin_W, lg, lb):
    full = lambda i: (0, 0)
    blk = lambda i: (i, 0)
    return pl.pallas_call(
        _edge_body,
        grid=(N_AGT // AB,),
        in_specs=[
            pl.BlockSpec((AB, K, D), lambda i: (i, 0, 0)),
            pl.BlockSpec((AB * K, 1), blk),
            pl.BlockSpec((AB * K, 1), blk),
            pl.BlockSpec((AB, D), blk),
            pl.BlockSpec((AB, D), blk),
            pl.BlockSpec((1, D), full),
            pl.BlockSpec((1, D), full),
            pl.BlockSpec((1, D), full),
            pl.BlockSpec((D, D), full),
            pl.BlockSpec((1, D), full),
            pl.BlockSpec((1, D), full),
            pl.BlockSpec((D, D), full),
            pl.BlockSpec((1, D), full),
            pl.BlockSpec((1, D), full),
            pl.BlockSpec((D, D), full),
            pl.BlockSpec((D, D), full),
            pl.BlockSpec((1, D), full),
            pl.BlockSpec((1, D), full),
            pl.BlockSpec((D, D), full),
            pl.BlockSpec((1, D), full),
            pl.BlockSpec((1, D), full),
        ],
        out_specs=pl.BlockSpec((AB, D), blk),
        out_shape=jax.ShapeDtypeStruct((N_AGT, D), jnp.float32),
        scratch_shapes=[pltpu.VMEM((AB * K, AB), jnp.float32)],
    )(ef, dvx.reshape(N_AGT * K, 1), dvy.reshape(N_AGT * K, 1),
      qpart, agts,
      w1x, w1y, b1d, dist_W2, dg2, db2, W1d, cg1, cb1, ctx_W2,
      agt_W, ng, nb, lin_W, lg, lb)


def kernel(agts, agt_idcs, agt_ctrs, ctx, ctx_idcs, ctx_ctrs, dist_th,
           dist_W1, dist_b1, dist_W2, dist_g2, dist_b2,
           q_W, q_g, q_b, ctx_W1, ctx_g1, ctx_b1, ctx_W2,
           agt_W, norm_g, norm_b, lin_W, lin_g, lin_b):
    f32 = jnp.float32
    th = jnp.asarray(dist_th, f32)
    th2 = jnp.nextafter(th * th, jnp.asarray(jnp.inf, f32))
    th2v = jnp.broadcast_to(th2, (16,))

    ctx_x = ctx_ctrs[:, 0]
    ctx_y = ctx_ctrs[:, 1]
    agt_x = agt_ctrs[:, 0]
    agt_y = agt_ctrs[:, 1]

    W1d = ctx_W1[:, :D]
    W1q = ctx_W1[:, D:2 * D]
    W1c = ctx_W1[:, 2 * D:]
    w1x = dist_W1[:, 0].reshape(1, D)
    w1y = dist_W1[:, 1].reshape(1, D)

    qpart, cpart = _run_prework(agts, q_W, q_g, q_b, W1q, ctx, W1c)
    dvx, dvy, ef = _sc_search_gather(ctx_x, ctx_y, agt_x, agt_y,
                                     th2v, cpart)
    return _run_edge(
        ef, dvx, dvy, qpart, agts,
        w1x, w1y, dist_b1.reshape(1, D), dist_W2,
        dist_g2.reshape(1, D), dist_b2.reshape(1, D),
        W1d, ctx_g1.reshape(1, D), ctx_b1.reshape(1, D), ctx_W2,
        agt_W, norm_g.reshape(1, D), norm_b.reshape(1, D),
        lin_W, lin_g.reshape(1, D), lin_b.reshape(1, D))


# speedup vs baseline: 1.0566x; 1.0566x over previous
import functools

import jax
import jax.numpy as jnp
from jax import lax
from jax.experimental import pallas as pl
from jax.experimental.pallas import tpu as pltpu
from jax.experimental.pallas import tpu_sc as plsc

N_AGT = 4096
N_CTX = 8192
D = 128
K = 128
SLACK = 16
AB = 64
EPS = 1e-5
SENT = 1e9


def _gn_rows(x, g, b):
    m = jnp.mean(x, axis=-1, keepdims=True)
    ms = jnp.mean(x * x, axis=-1, keepdims=True)
    v = ms - m * m
    return (x - m) * lax.rsqrt(v + EPS) * g + b


def _qpart_body(agts_ref, qW_ref, qg_ref, qb_ref, W1q_ref, o_ref):
    x = agts_ref[...]
    q = lax.dot_general(x, qW_ref[...], (((1,), (1,)), ((), ())))
    q = jnp.maximum(_gn_rows(q, qg_ref[...], qb_ref[...]), 0.0)
    o_ref[...] = lax.dot_general(q, W1q_ref[...], (((1,), (1,)), ((), ())))


def _cpart_body(ctx_ref, W1c_ref, o_ref):
    o_ref[...] = lax.dot_general(ctx_ref[...], W1c_ref[...],
                                 (((1,), (1,)), ((), ())))


def _run_prework(agts, q_W, q_g, q_b, W1q, ctx, W1c):
    rb = min(1024, N_AGT, N_CTX)
    full = lambda i: (0, 0)
    qpart = pl.pallas_call(
        _qpart_body,
        grid=(N_AGT // rb,),
        in_specs=[
            pl.BlockSpec((rb, D), lambda i: (i, 0)),
            pl.BlockSpec((D, D), full),
            pl.BlockSpec((1, D), full),
            pl.BlockSpec((1, D), full),
            pl.BlockSpec((D, D), full),
        ],
        out_specs=pl.BlockSpec((rb, D), lambda i: (i, 0)),
        out_shape=jax.ShapeDtypeStruct((N_AGT, D), jnp.float32),
    )(agts, q_W, q_g.reshape(1, D), q_b.reshape(1, D), W1q)
    cpart = pl.pallas_call(
        _cpart_body,
        grid=(N_CTX // rb,),
        in_specs=[
            pl.BlockSpec((rb, D), lambda i: (i, 0)),
            pl.BlockSpec((D, D), full),
        ],
        out_specs=pl.BlockSpec((rb, D), lambda i: (i, 0)),
        out_shape=jax.ShapeDtypeStruct((N_CTX, D), jnp.float32),
    )(ctx, W1c)
    return qpart, cpart


def _sc_search_gather(ctx_x, ctx_y, agt_x, agt_y, th2v, cpart):
    info = plsc.get_sparse_core_info()
    NC, NS = info.num_cores, info.num_subcores
    NW = NC * NS
    A_PER = N_AGT // NW

    mesh = plsc.VectorSubcoreMesh(core_axis_name="c", subcore_axis_name="s")

    @functools.partial(
        pl.kernel,
        out_type=(
            jax.ShapeDtypeStruct((N_AGT, K), jnp.float32),
            jax.ShapeDtypeStruct((N_AGT, K), jnp.float32),
            jax.ShapeDtypeStruct((N_AGT, K, D), jnp.float32),
        ),
        mesh=mesh,
        compiler_params=pltpu.CompilerParams(needs_layout_passes=False),
        scratch_types=[
            pltpu.VMEM((N_CTX,), jnp.float32),
            pltpu.VMEM((N_CTX,), jnp.float32),
            pltpu.VMEM((A_PER,), jnp.float32),
            pltpu.VMEM((A_PER,), jnp.float32),
            pltpu.VMEM((16,), jnp.float32),
            pltpu.VMEM((K + SLACK,), jnp.int32),
            pltpu.VMEM((K,), jnp.int32),
            pltpu.VMEM((K + SLACK,), jnp.float32),
            pltpu.VMEM((K + SLACK,), jnp.float32),
            pltpu.VMEM((K, D), jnp.float32),
            pltpu.VMEM_SHARED((N_CTX, D), jnp.float32),
            pltpu.SemaphoreType.DMA,
        ],
    )
    def body(ctx_x_h, ctx_y_h, agt_x_h, agt_y_h, th2_h, cpart_h,
             dvx_h, dvy_h, ef_h,
             cx, cy, ax, ay, th2s, idxb, idx2, dvxb, dvyb, rows,
             shared, sem):
        sid = lax.axis_index("s")
        wid = sid * NC + lax.axis_index("c")
        base = wid * A_PER
        sl = N_CTX // NS
        pltpu.sync_copy(cpart_h.at[pl.ds(sid * sl, sl)],
                        shared.at[pl.ds(sid * sl, sl)])
        pltpu.sync_copy(ctx_x_h, cx)
        pltpu.sync_copy(ctx_y_h, cy)
        pltpu.sync_copy(agt_x_h.at[pl.ds(base, A_PER)], ax)
        pltpu.sync_copy(agt_y_h.at[pl.ds(base, A_PER)], ay)
        pltpu.sync_copy(th2_h, th2s)
        plsc.subcore_barrier()
        th2 = th2s[...]
        lanes = lax.iota(jnp.int32, 16)
        zf = jnp.zeros((16,), jnp.float32)
        zi = jnp.zeros((16,), jnp.int32)
        sentinel = jnp.full((16,), SENT, jnp.float32)

        def per_agent(a, carry):
            for t in range(K // 16):
                idxb[pl.ds(t * 16, 16)] = zi
                dvxb[pl.ds(t * 16, 16)] = sentinel
            a0 = (a // 16) * 16
            lane = a - a0
            axs = jnp.sum(jnp.where(lanes == lane, ax[pl.ds(a0, 16)], zf))
            ays = jnp.sum(jnp.where(lanes == lane, ay[pl.ds(a0, 16)], zf))
            axb = jnp.full((16,), axs)
            ayb = jnp.full((16,), ays)

            @plsc.parallel_loop(0, N_CTX // 16, unroll=4, carry=zi)
            def _chunks(c, o):
                dx = axb - cx[pl.ds(c * 16, 16)]
                dy = ayb - cy[pl.ds(c * 16, 16)]
                m = dx * dx + dy * dy <= th2
                cum = plsc.cumsum(m.astype(jnp.int32))
                pos = jnp.clip(o + cum - 1, 0, K + SLACK - 1)
                plsc.store_scatter(idxb, [pos], c * 16 + lanes, mask=m)
                plsc.store_scatter(dvxb, [pos], dx, mask=m)
                plsc.store_scatter(dvyb, [pos], dy, mask=m)
                return o + plsc.all_reduce_population_count(m)

            for t in range(K // 16):
                idx2[pl.ds(t * 16, 16)] = idxb[pl.ds(t * 16, 16)]
            g = base + a
            pltpu.async_copy(shared.at[idx2], rows, sem).wait()
            pltpu.sync_copy(rows, ef_h.at[g])
            pltpu.sync_copy(dvxb.at[pl.ds(0, K)], dvx_h.at[g])
            pltpu.sync_copy(dvyb.at[pl.ds(0, K)], dvy_h.at[g])
            return carry

        lax.fori_loop(0, A_PER, per_agent, 0)

    return body(ctx_x, ctx_y, agt_x, agt_y, th2v, cpart)


def _edge_body(ef_ref, dvx_ref, dvy_ref, qp_ref, agts_ref,
               w1x_ref, w1y_ref, b1d_ref, dW2_ref, dg2_ref, db2_ref,
               W1d_ref, cg1_ref, cb1_ref, cW2_ref,
               aW_ref, ng_ref, nb_ref, lW_ref, lg_ref, lb_ref, o_ref):
    R = AB * K
    dvx = dvx_ref[...]
    dvy = dvy_ref[...]
    d1 = jnp.maximum(dvx * w1x_ref[...] + dvy * w1y_ref[...] + b1d_ref[...],
                     0.0)
    d2 = lax.dot_general(d1, dW2_ref[...], (((1,), (1,)), ((), ())))
    d2 = jnp.maximum(_gn_rows(d2, dg2_ref[...], db2_ref[...]), 0.0)
    z = lax.dot_general(d2, W1d_ref[...], (((1,), (1,)), ((), ())))
    z = z + ef_ref[...].reshape(R, D)
    z = z + jnp.broadcast_to(qp_ref[...][:, None, :], (AB, K, D)).reshape(R, D)
    h = jnp.maximum(_gn_rows(z, cg1_ref[...], cb1_ref[...]), 0.0)
    e = lax.dot_general(h, cW2_ref[...], (((1,), (1,)), ((), ())))
    e = jnp.where(dvx < SENT * 0.5, e, 0.0)
    msgs = e.reshape(AB, K, D).sum(axis=1)
    res = agts_ref[...]
    a = lax.dot_general(res, aW_ref[...], (((1,), (1,)), ((), ()))) + msgs
    a = jnp.maximum(_gn_rows(a, ng_ref[...], nb_ref[...]), 0.0)
    a = lax.dot_general(a, lW_ref[...], (((1,), (1,)), ((), ())))
    a = _gn_rows(a, lg_ref[...], lb_ref[...])
    o_ref[...] = jnp.maximum(a + res, 0.0)


def _run_edge(ef, dvx, dvy, qpart, agts,
              w1x, w1y, b1d, dist_W2, dg2, db2,
              W1d, cg1, cb1, ctx_W2, agt_W, ng, nb, lin_W, lg, lb):
    full = lambda i: (0, 0)
    blk = lambda i: (i, 0)
    return pl.pallas_call(
        _edge_body,
        grid=(N_AGT // AB,),
        in_specs=[
            pl.BlockSpec((AB, K, D), lambda i: (i, 0, 0)),
            pl.BlockSpec((AB * K, 1), blk),
            pl.BlockSpec((AB * K, 1), blk),
            pl.BlockSpec((AB, D), blk),
            pl.BlockSpec((AB, D), blk),
            pl.BlockSpec((1, D), full),
            pl.BlockSpec((1, D), full),
            pl.BlockSpec((1, D), full),
            pl.BlockSpec((D, D), full),
            pl.BlockSpec((1, D), full),
            pl.BlockSpec((1, D), full),
            pl.BlockSpec((D, D), full),
            pl.BlockSpec((1, D), full),
            pl.BlockSpec((1, D), full),
            pl.BlockSpec((D, D), full),
            pl.BlockSpec((D, D), full),
            pl.BlockSpec((1, D), full),
            pl.BlockSpec((1, D), full),
            pl.BlockSpec((D, D), full),
            pl.BlockSpec((1, D), full),
            pl.BlockSpec((1, D), full),
        ],
        out_specs=pl.BlockSpec((AB, D), blk),
        out_shape=jax.ShapeDtypeStruct((N_AGT, D), jnp.float32),
    )(ef, dvx.reshape(N_AGT * K, 1), dvy.reshape(N_AGT * K, 1),
      qpart, agts,
      w1x, w1y, b1d, dist_W2, dg2, db2, W1d, cg1, cb1, ctx_W2,
      agt_W, ng, nb, lin_W, lg, lb)


def kernel(agts, agt_idcs, agt_ctrs, ctx, ctx_idcs, ctx_ctrs, dist_th,
           dist_W1, dist_b1, dist_W2, dist_g2, dist_b2,
           q_W, q_g, q_b, ctx_W1, ctx_g1, ctx_b1, ctx_W2,
           agt_W, norm_g, norm_b, lin_W, lin_g, lin_b):
    f32 = jnp.float32
    th = jnp.asarray(dist_th, f32)
    th2 = jnp.nextafter(th * th, jnp.asarray(jnp.inf, f32))
    th2v = jnp.broadcast_to(th2, (16,))

    ctx_x = ctx_ctrs[:, 0]
    ctx_y = ctx_ctrs[:, 1]
    agt_x = agt_ctrs[:, 0]
    agt_y = agt_ctrs[:, 1]

    W1d = ctx_W1[:, :D]
    W1q = ctx_W1[:, D:2 * D]
    W1c = ctx_W1[:, 2 * D:]
    w1x = dist_W1[:, 0].reshape(1, D)
    w1y = dist_W1[:, 1].reshape(1, D)

    qpart, cpart = _run_prework(agts, q_W, q_g, q_b, W1q, ctx, W1c)
    dvx, dvy, ef = _sc_search_gather(ctx_x, ctx_y, agt_x, agt_y,
                                     th2v, cpart)
    return _run_edge(
        ef, dvx, dvy, qpart, agts,
        w1x, w1y, dist_b1.reshape(1, D), dist_W2,
        dist_g2.reshape(1, D), dist_b2.reshape(1, D),
        W1d, ctx_g1.reshape(1, D), ctx_b1.reshape(1, D), ctx_W2,
        agt_W, norm_g.reshape(1, D), norm_b.reshape(1, D),
        lin_W, lin_g.reshape(1, D), lin_b.reshape(1, D))


# speedup vs baseline: 1.2722x; 1.2040x over previous
import functools

import jax
import jax.numpy as jnp
from jax import lax
from jax.experimental import pallas as pl
from jax.experimental.pallas import tpu as pltpu
from jax.experimental.pallas import tpu_sc as plsc

N_AGT = 4096
N_CTX = 8192
D = 128
K = 128
SLACK = 16
AB = 64
EPS = 1e-5
SENT = 1e9


def _gn_rows(x, g, b):
    m = jnp.mean(x, axis=-1, keepdims=True)
    v = jnp.mean((x - m) ** 2, axis=-1, keepdims=True)
    return (x - m) * lax.rsqrt(v + EPS) * g + b


def _qpart_body(agts_ref, qW_ref, qg_ref, qb_ref, W1q_ref, o_ref):
    x = agts_ref[...]
    q = lax.dot_general(x, qW_ref[...], (((1,), (1,)), ((), ())))
    q = jnp.maximum(_gn_rows(q, qg_ref[...], qb_ref[...]), 0.0)
    o_ref[...] = lax.dot_general(q, W1q_ref[...], (((1,), (1,)), ((), ())))


def _cpart_body(ctx_ref, W1c_ref, o_ref):
    o_ref[...] = lax.dot_general(ctx_ref[...], W1c_ref[...],
                                 (((1,), (1,)), ((), ())))


def _run_prework(agts, q_W, q_g, q_b, W1q, ctx, W1c):
    rb = min(1024, N_AGT, N_CTX)
    full = lambda i: (0, 0)
    qpart = pl.pallas_call(
        _qpart_body,
        grid=(N_AGT // rb,),
        in_specs=[
            pl.BlockSpec((rb, D), lambda i: (i, 0)),
            pl.BlockSpec((D, D), full),
            pl.BlockSpec((1, D), full),
            pl.BlockSpec((1, D), full),
            pl.BlockSpec((D, D), full),
        ],
        out_specs=pl.BlockSpec((rb, D), lambda i: (i, 0)),
        out_shape=jax.ShapeDtypeStruct((N_AGT, D), jnp.float32),
    )(agts, q_W, q_g.reshape(1, D), q_b.reshape(1, D), W1q)
    cpart = pl.pallas_call(
        _cpart_body,
        grid=(N_CTX // rb,),
        in_specs=[
            pl.BlockSpec((rb, D), lambda i: (i, 0)),
            pl.BlockSpec((D, D), full),
        ],
        out_specs=pl.BlockSpec((rb, D), lambda i: (i, 0)),
        out_shape=jax.ShapeDtypeStruct((N_CTX, D), jnp.float32),
    )(ctx, W1c)
    return qpart, cpart


def _sc_search_gather(ctx_x, ctx_y, agt_x, agt_y, th2v, cpart):
    info = plsc.get_sparse_core_info()
    NC, NS = info.num_cores, info.num_subcores
    NW = NC * NS
    A_PER = N_AGT // NW

    mesh = plsc.VectorSubcoreMesh(core_axis_name="c", subcore_axis_name="s")

    @functools.partial(
        pl.kernel,
        out_type=(
            jax.ShapeDtypeStruct((N_AGT, K), jnp.float32),
            jax.ShapeDtypeStruct((N_AGT, K), jnp.float32),
            jax.ShapeDtypeStruct((N_AGT, K, D), jnp.float32),
        ),
        mesh=mesh,
        compiler_params=pltpu.CompilerParams(needs_layout_passes=False),
        scratch_types=[
            pltpu.VMEM((N_CTX,), jnp.float32),
            pltpu.VMEM((N_CTX,), jnp.float32),
            pltpu.VMEM((A_PER,), jnp.float32),
            pltpu.VMEM((A_PER,), jnp.float32),
            pltpu.VMEM((16,), jnp.float32),
            pltpu.VMEM((K,), jnp.int32),
            pltpu.VMEM((K,), jnp.int32),
            pltpu.VMEM((K,), jnp.float32),
            pltpu.VMEM((K,), jnp.float32),
            pltpu.VMEM((K, D), jnp.float32),
            pltpu.VMEM((K, D), jnp.float32),
            pltpu.VMEM_SHARED((N_CTX, D), jnp.float32),
            pltpu.SemaphoreType.DMA,
            pltpu.SemaphoreType.DMA,
            pltpu.SemaphoreType.DMA,
            pltpu.SemaphoreType.DMA,
        ],
    )
    def body(ctx_x_h, ctx_y_h, agt_x_h, agt_y_h, th2_h, cpart_h,
             dvx_h, dvy_h, ef_h,
             cx, cy, ax, ay, th2s, idx0, idx1, dvxb, dvyb, rows0, rows1,
             shared, g0, g1, w0, w1):
        sid = lax.axis_index("s")
        wid = sid * NC + lax.axis_index("c")
        base = wid * A_PER
        sl = N_CTX // NS
        pltpu.sync_copy(cpart_h.at[pl.ds(sid * sl, sl)],
                        shared.at[pl.ds(sid * sl, sl)])
        pltpu.sync_copy(ctx_x_h, cx)
        pltpu.sync_copy(ctx_y_h, cy)
        pltpu.sync_copy(agt_x_h.at[pl.ds(base, A_PER)], ax)
        pltpu.sync_copy(agt_y_h.at[pl.ds(base, A_PER)], ay)
        pltpu.sync_copy(th2_h, th2s)
        plsc.subcore_barrier()
        th2 = th2s[...]
        lanes = lax.iota(jnp.int32, 16)
        zf = jnp.zeros((16,), jnp.float32)
        zi = jnp.zeros((16,), jnp.int32)
        sentinel = jnp.full((16,), SENT, jnp.float32)

        def search(a, idxp):
            for t in range(K // 16):
                idxp[pl.ds(t * 16, 16)] = zi
                dvxb[pl.ds(t * 16, 16)] = sentinel
            a0 = (a // 16) * 16
            lane = a - a0
            axs = jnp.sum(jnp.where(lanes == lane, ax[pl.ds(a0, 16)], zf))
            ays = jnp.sum(jnp.where(lanes == lane, ay[pl.ds(a0, 16)], zf))
            axb = jnp.full((16,), axs)
            ayb = jnp.full((16,), ays)

            @plsc.parallel_loop(0, N_CTX // 16, unroll=4, carry=zi)
            def _chunks(c, o):
                dx = axb - cx[pl.ds(c * 16, 16)]
                dy = ayb - cy[pl.ds(c * 16, 16)]
                m = dx * dx + dy * dy <= th2
                cum = plsc.cumsum(m.astype(jnp.int32))
                pos = jnp.clip(o + cum - 1, 0, K - 1)
                plsc.store_scatter(idxp, [pos], c * 16 + lanes, mask=m)
                plsc.store_scatter(dvxb, [pos], dx, mask=m)
                plsc.store_scatter(dvyb, [pos], dy, mask=m)
                return o + plsc.all_reduce_population_count(m)

            pltpu.sync_copy(dvxb, dvx_h.at[base + a])
            pltpu.sync_copy(dvyb, dvy_h.at[base + a])

        def fire_gather(idxp, rows, sem):
            return pltpu.async_copy(shared.at[idxp], rows, sem)

        def fire_wb(a, rows, sem):
            return pltpu.async_copy(rows, ef_h.at[base + a], sem)

        def wait_gather(rows, sem):
            pltpu.make_async_copy(cpart_h.at[pl.ds(0, K)], rows, sem).wait()

        def wait_wb(rows, sem):
            pltpu.make_async_copy(rows, ef_h.at[base], sem).wait()

        search(0, idx0)
        fire_gather(idx0, rows0, g0)
        search(1, idx1)
        fire_gather(idx1, rows1, g1)
        wait_gather(rows0, g0)
        fire_wb(0, rows0, w0)

        def pair(i, carry):
            a0 = 2 * i
            a1 = a0 + 1
            search(a0, idx0)
            wait_gather(rows1, g1)
            fire_wb(a1 - 2, rows1, w1)
            wait_wb(rows0, w0)
            fire_gather(idx0, rows0, g0)
            search(a1, idx1)
            wait_gather(rows0, g0)
            fire_wb(a0, rows0, w0)
            wait_wb(rows1, w1)
            fire_gather(idx1, rows1, g1)
            return carry

        lax.fori_loop(1, A_PER // 2, pair, 0)
        wait_gather(rows1, g1)
        fire_wb(A_PER - 1, rows1, w1)
        wait_wb(rows0, w0)
        wait_wb(rows1, w1)

    return body(ctx_x, ctx_y, agt_x, agt_y, th2v, cpart)


def _edge_body(ef_ref, dvx_ref, dvy_ref, qp_ref, agts_ref,
               w1x_ref, w1y_ref, b1d_ref, dW2_ref, dg2_ref, db2_ref,
               W1d_ref, cg1_ref, cb1_ref, cW2_ref,
               aW_ref, ng_ref, nb_ref, lW_ref, lg_ref, lb_ref, o_ref):
    R = AB * K
    dvx = dvx_ref[...]
    dvy = dvy_ref[...]
    d1 = jnp.maximum(dvx * w1x_ref[...] + dvy * w1y_ref[...] + b1d_ref[...],
                     0.0)
    d2 = lax.dot_general(d1, dW2_ref[...], (((1,), (1,)), ((), ())))
    d2 = jnp.maximum(_gn_rows(d2, dg2_ref[...], db2_ref[...]), 0.0)
    z = lax.dot_general(d2, W1d_ref[...], (((1,), (1,)), ((), ())))
    z = z + ef_ref[...].reshape(R, D)
    z = z + jnp.broadcast_to(qp_ref[...][:, None, :], (AB, K, D)).reshape(R, D)
    h = jnp.maximum(_gn_rows(z, cg1_ref[...], cb1_ref[...]), 0.0)
    e = lax.dot_general(h, cW2_ref[...], (((1,), (1,)), ((), ())))
    e = jnp.where(dvx < SENT * 0.5, e, 0.0)
    msgs = e.reshape(AB, K, D).sum(axis=1)
    res = agts_ref[...]
    a = lax.dot_general(res, aW_ref[...], (((1,), (1,)), ((), ()))) + msgs
    a = jnp.maximum(_gn_rows(a, ng_ref[...], nb_ref[...]), 0.0)
    a = lax.dot_general(a, lW_ref[...], (((1,), (1,)), ((), ())))
    a = _gn_rows(a, lg_ref[...], lb_ref[...])
    o_ref[...] = jnp.maximum(a + res, 0.0)


def _run_edge(ef, dvx, dvy, qpart, agts,
              w1x, w1y, b1d, dist_W2, dg2, db2,
              W1d, cg1, cb1, ctx_W2, agt_W, ng, nb, lin_W, lg, lb):
    full = lambda i: (0, 0)
    blk = lambda i: (i, 0)
    return pl.pallas_call(
        _edge_body,
        grid=(N_AGT // AB,),
        in_specs=[
            pl.BlockSpec((AB, K, D), lambda i: (i, 0, 0)),
            pl.BlockSpec((AB * K, 1), blk),
            pl.BlockSpec((AB * K, 1), blk),
            pl.BlockSpec((AB, D), blk),
            pl.BlockSpec((AB, D), blk),
            pl.BlockSpec((1, D), full),
            pl.BlockSpec((1, D), full),
            pl.BlockSpec((1, D), full),
            pl.BlockSpec((D, D), full),
            pl.BlockSpec((1, D), full),
            pl.BlockSpec((1, D), full),
            pl.BlockSpec((D, D), full),
            pl.BlockSpec((1, D), full),
            pl.BlockSpec((1, D), full),
            pl.BlockSpec((D, D), full),
            pl.BlockSpec((D, D), full),
            pl.BlockSpec((1, D), full),
            pl.BlockSpec((1, D), full),
            pl.BlockSpec((D, D), full),
            pl.BlockSpec((1, D), full),
            pl.BlockSpec((1, D), full),
        ],
        out_specs=pl.BlockSpec((AB, D), blk),
        out_shape=jax.ShapeDtypeStruct((N_AGT, D), jnp.float32),
    )(ef, dvx.reshape(N_AGT * K, 1), dvy.reshape(N_AGT * K, 1),
      qpart, agts,
      w1x, w1y, b1d, dist_W2, dg2, db2, W1d, cg1, cb1, ctx_W2,
      agt_W, ng, nb, lin_W, lg, lb)


def kernel(agts, agt_idcs, agt_ctrs, ctx, ctx_idcs, ctx_ctrs, dist_th,
           dist_W1, dist_b1, dist_W2, dist_g2, dist_b2,
           q_W, q_g, q_b, ctx_W1, ctx_g1, ctx_b1, ctx_W2,
           agt_W, norm_g, norm_b, lin_W, lin_g, lin_b):
    f32 = jnp.float32
    th = jnp.asarray(dist_th, f32)
    th2 = jnp.nextafter(th * th, jnp.asarray(jnp.inf, f32))
    th2v = jnp.broadcast_to(th2, (16,))

    ctx_x = ctx_ctrs[:, 0]
    ctx_y = ctx_ctrs[:, 1]
    agt_x = agt_ctrs[:, 0]
    agt_y = agt_ctrs[:, 1]

    W1d = ctx_W1[:, :D]
    W1q = ctx_W1[:, D:2 * D]
    W1c = ctx_W1[:, 2 * D:]
    w1x = dist_W1[:, 0].reshape(1, D)
    w1y = dist_W1[:, 1].reshape(1, D)

    qpart, cpart = _run_prework(agts, q_W, q_g, q_b, W1q, ctx, W1c)
    dvx, dvy, ef = _sc_search_gather(ctx_x, ctx_y, agt_x, agt_y,
                                     th2v, cpart)
    return _run_edge(
        ef, dvx, dvy, qpart, agts,
        w1x, w1y, dist_b1.reshape(1, D), dist_W2,
        dist_g2.reshape(1, D), dist_b2.reshape(1, D),
        W1d, ctx_g1.reshape(1, D), ctx_b1.reshape(1, D), ctx_W2,
        agt_W, norm_g.reshape(1, D), norm_b.reshape(1, D),
        lin_W, lin_g.reshape(1, D), lin_b.reshape(1, D))
